# initial kernel scaffold (unmeasured)
import jax
import jax.numpy as jnp
from jax import lax
from jax.experimental import pallas as pl
from jax.experimental.pallas import tpu as pltpu

N_DEV = 4
TH = 1024


def kernel(x, Win0, Wout0, Win1, Wout1, Win2, Wout2):
    B, D = x.shape
    H = Win0.shape[1]
    NT = H // TH
    NL = 3
    Bq = B // N_DEV

    def body(x_ref, win0, wout0, win1, wout1, win2, wout2, out_ref,
             xbuf, ybuf, win_vm, wout_vm, ar_recv, rs_recv,
             load_sems, ar_send_sems, ar_recv_sems, rs_send_sems, rs_recv_sems):
        me = lax.axis_index("i")
        wins = [win0, win1, win2]
        wouts = [wout0, wout1, wout2]

        barrier = pltpu.get_barrier_semaphore()
        for d in range(1, N_DEV):
            pl.semaphore_signal(
                barrier, inc=1,
                device_id=((me + d) % N_DEV,),
                device_id_type=pl.DeviceIdType.MESH,
            )
        pl.semaphore_wait(barrier, N_DEV - 1)

        def start_load(g):
            l, t = divmod(g, NT)
            slot = g % 2
            cw = pltpu.make_async_copy(
                wins[l].at[:, pl.ds(t * TH, TH)], win_vm.at[slot],
                load_sems.at[0, slot])
            co = pltpu.make_async_copy(
                wouts[l].at[pl.ds(t * TH, TH), :], wout_vm.at[slot],
                load_sems.at[1, slot])
            cw.start()
            co.start()
            return cw, co

        def ar_descs(l):
            return [
                pltpu.make_async_remote_copy(
                    src_ref=ybuf,
                    dst_ref=ar_recv.at[l, d - 1],
                    send_sem=ar_send_sems.at[l, d - 1],
                    recv_sem=ar_recv_sems.at[l, d - 1],
                    device_id=((me + d) % N_DEV,),
                    device_id_type=pl.DeviceIdType.MESH,
                )
                for d in range(1, N_DEV)
            ]

        pending = {0: start_load(0)}
        ar = {}
        for g in range(NL * NT):
            l, t = divmod(g, NT)
            slot = g % 2
            if g + 1 < NL * NT:
                pending[g + 1] = start_load(g + 1)
            if t == 0 and l > 0:
                for r in ar[l - 1]:
                    r.wait_send()
                for r in ar[l - 1]:
                    r.wait_recv()
                xbuf[...] = (ybuf[...] + ar_recv[l - 1, 0]
                             + ar_recv[l - 1, 1] + ar_recv[l - 1, 2])
            cw, co = pending.pop(g)
            cw.wait()
            co.wait()
            xin = x_ref[...] if l == 0 else xbuf[...]
            h = jnp.maximum(
                jnp.dot(xin, win_vm[slot], preferred_element_type=jnp.float32),
                0.0)
            yp = jnp.dot(h, wout_vm[slot], preferred_element_type=jnp.float32)
            if t == 0:
                ybuf[...] = yp
            else:
                ybuf[...] = ybuf[...] + yp
            if t == NT - 1:
                if l < NL - 1:
                    descs = ar_descs(l)
                    for r in descs:
                        r.start()
                    ar[l] = descs
                else:
                    rs = []
                    for d in range(1, N_DEV):
                        j = (me + d) % N_DEV
                        r = pltpu.make_async_remote_copy(
                            src_ref=ybuf.at[pl.ds(j * Bq, Bq), :],
                            dst_ref=rs_recv.at[d - 1],
                            send_sem=rs_send_sems.at[d - 1],
                            recv_sem=rs_recv_sems.at[d - 1],
                            device_id=(j,),
                            device_id_type=pl.DeviceIdType.MESH,
                        )
                        r.start()
                        rs.append(r)
                    for r in rs:
                        r.wait_send()
                    for r in rs:
                        r.wait_recv()
                    out_ref[...] = (ybuf[pl.ds(me * Bq, Bq), :]
                                    + rs_recv[0] + rs_recv[1] + rs_recv[2])

    return pl.pallas_call(
        body,
        out_shape=jax.ShapeDtypeStruct((Bq, D), jnp.float32),
        in_specs=[
            pl.BlockSpec(memory_space=pltpu.VMEM),
            pl.BlockSpec(memory_space=pltpu.ANY),
            pl.BlockSpec(memory_space=pltpu.ANY),
            pl.BlockSpec(memory_space=pltpu.ANY),
            pl.BlockSpec(memory_space=pltpu.ANY),
            pl.BlockSpec(memory_space=pltpu.ANY),
            pl.BlockSpec(memory_space=pltpu.ANY),
        ],
        out_specs=pl.BlockSpec(memory_space=pltpu.VMEM),
        scratch_shapes=[
            pltpu.VMEM((B, D), jnp.float32),
            pltpu.VMEM((B, D), jnp.float32),
            pltpu.VMEM((2, D, TH), jnp.float32),
            pltpu.VMEM((2, TH, D), jnp.float32),
            pltpu.VMEM((2, 3, B, D), jnp.float32),
            pltpu.VMEM((3, Bq, D), jnp.float32),
            pltpu.SemaphoreType.DMA((2, 2)),
            pltpu.SemaphoreType.DMA((2, 3)),
            pltpu.SemaphoreType.DMA((2, 3)),
            pltpu.SemaphoreType.DMA((3,)),
            pltpu.SemaphoreType.DMA((3,)),
        ],
        compiler_params=pltpu.CompilerParams(collective_id=0),
    )(x, Win0, Wout0, Win1, Wout1, Win2, Wout2)


# baseline (device time: 73161 ns/iter reference)
import jax
import jax.numpy as jnp
from jax import lax
from jax.experimental import pallas as pl
from jax.experimental.pallas import tpu as pltpu

N_DEV = 4
TH = 1024


def kernel(x, Win0, Wout0, Win1, Wout1, Win2, Wout2):
    B, D = x.shape
    H = Win0.shape[1]
    NT = H // TH
    NL = 3
    Bq = B // N_DEV

    def body(x_ref, win0, wout0, win1, wout1, win2, wout2, out_ref,
             xbuf, ybuf, win_vm, wout_vm, ar_recv, rs_recv,
             load_sems, ar_send_sems, ar_recv_sems, rs_send_sems, rs_recv_sems):
        me = lax.axis_index("i")
        wins = [win0, win1, win2]
        wouts = [wout0, wout1, wout2]

        barrier = pltpu.get_barrier_semaphore()
        for d in range(1, N_DEV):
            pl.semaphore_signal(
                barrier, inc=1,
                device_id=((me + d) % N_DEV,),
                device_id_type=pl.DeviceIdType.MESH,
            )
        pl.semaphore_wait(barrier, N_DEV - 1)

        def start_load(g):
            l, t = divmod(g, NT)
            slot = g % 2
            cw = pltpu.make_async_copy(
                wins[l].at[:, pl.ds(t * TH, TH)], win_vm.at[slot],
                load_sems.at[0, slot])
            co = pltpu.make_async_copy(
                wouts[l].at[pl.ds(t * TH, TH), :], wout_vm.at[slot],
                load_sems.at[1, slot])
            cw.start()
            co.start()
            return cw, co

        def ar_descs(l):
            return [
                pltpu.make_async_remote_copy(
                    src_ref=ybuf,
                    dst_ref=ar_recv.at[l, d - 1],
                    send_sem=ar_send_sems.at[l, d - 1],
                    recv_sem=ar_recv_sems.at[l, d - 1],
                    device_id=((me + d) % N_DEV,),
                    device_id_type=pl.DeviceIdType.MESH,
                )
                for d in range(1, N_DEV)
            ]

        pending = {0: start_load(0)}
        ar = {}
        for g in range(NL * NT):
            l, t = divmod(g, NT)
            slot = g % 2
            if g + 1 < NL * NT:
                pending[g + 1] = start_load(g + 1)
            if t == 0 and l > 0:
                for r in ar[l - 1]:
                    r.wait_send()
                for r in ar[l - 1]:
                    r.wait_recv()
                xbuf[...] = (ybuf[...] + ar_recv[l - 1, 0]
                             + ar_recv[l - 1, 1] + ar_recv[l - 1, 2])
            cw, co = pending.pop(g)
            cw.wait()
            co.wait()
            xin = x_ref[...] if l == 0 else xbuf[...]
            h = jnp.maximum(
                jnp.dot(xin, win_vm[slot], preferred_element_type=jnp.float32),
                0.0)
            yp = jnp.dot(h, wout_vm[slot], preferred_element_type=jnp.float32)
            if t == 0:
                ybuf[...] = yp
            else:
                ybuf[...] = ybuf[...] + yp
            if t == NT - 1:
                if l < NL - 1:
                    descs = ar_descs(l)
                    for r in descs:
                        r.start()
                    ar[l] = descs
                else:
                    rs = []
                    for d in range(1, N_DEV):
                        j = (me + d) % N_DEV
                        r = pltpu.make_async_remote_copy(
                            src_ref=ybuf.at[pl.ds(j * Bq, Bq), :],
                            dst_ref=rs_recv.at[d - 1],
                            send_sem=rs_send_sems.at[d - 1],
                            recv_sem=rs_recv_sems.at[d - 1],
                            device_id=(j,),
                            device_id_type=pl.DeviceIdType.MESH,
                        )
                        r.start()
                        rs.append(r)
                    for r in rs:
                        r.wait_send()
                    for r in rs:
                        r.wait_recv()
                    out_ref[...] = (ybuf[pl.ds(me * Bq, Bq), :]
                                    + rs_recv[0] + rs_recv[1] + rs_recv[2])

    return pl.pallas_call(
        body,
        out_shape=jax.ShapeDtypeStruct((Bq, D), jnp.float32),
        in_specs=[
            pl.BlockSpec(memory_space=pltpu.MemorySpace.VMEM),
            pl.BlockSpec(memory_space=pl.ANY),
            pl.BlockSpec(memory_space=pl.ANY),
            pl.BlockSpec(memory_space=pl.ANY),
            pl.BlockSpec(memory_space=pl.ANY),
            pl.BlockSpec(memory_space=pl.ANY),
            pl.BlockSpec(memory_space=pl.ANY),
        ],
        out_specs=pl.BlockSpec(memory_space=pltpu.MemorySpace.VMEM),
        scratch_shapes=[
            pltpu.VMEM((B, D), jnp.float32),
            pltpu.VMEM((B, D), jnp.float32),
            pltpu.VMEM((2, D, TH), jnp.float32),
            pltpu.VMEM((2, TH, D), jnp.float32),
            pltpu.VMEM((2, 3, B, D), jnp.float32),
            pltpu.VMEM((3, Bq, D), jnp.float32),
            pltpu.SemaphoreType.DMA((2, 2)),
            pltpu.SemaphoreType.DMA((2, 3)),
            pltpu.SemaphoreType.DMA((2, 3)),
            pltpu.SemaphoreType.DMA((3,)),
            pltpu.SemaphoreType.DMA((3,)),
        ],
        compiler_params=pltpu.CompilerParams(
            collective_id=0,
            vmem_limit_bytes=60 * 1024 * 1024,
        ),
    )(x, Win0, Wout0, Win1, Wout1, Win2, Wout2)


# device time: 72826 ns/iter; 1.0046x vs baseline; 1.0046x over previous
import jax
import jax.numpy as jnp
from jax import lax
from jax.experimental import pallas as pl
from jax.experimental.pallas import tpu as pltpu

N_DEV = 4
NKW = 4
NTO = 4
NL = 3


def kernel(x, Win0, Wout0, Win1, Wout1, Win2, Wout2):
    B, D = x.shape
    H = Win0.shape[1]
    DK = D // NKW
    TH = H // NTO
    Bq = B // N_DEV

    def body(x_ref, win0, wout0, win1, wout1, win2, wout2, out_ref,
             xbuf, ybuf, hbuf, ysend, win_vm, wout_vm, ar_recv, rs_recv,
             load_sems, ar_send_sems, ar_recv_sems, rs_send_sems, rs_recv_sems):
        me = lax.axis_index("i")
        wins = [win0, win1, win2]
        wouts = [wout0, wout1, wout2]

        barrier = pltpu.get_barrier_semaphore()
        for d in range(1, N_DEV):
            pl.semaphore_signal(
                barrier, inc=1,
                device_id=((me + d) % N_DEV,),
                device_id_type=pl.DeviceIdType.MESH,
            )
        pl.semaphore_wait(barrier, N_DEV - 1)

        def issue_win(wi):
            l, k = divmod(wi, NKW)
            slot = wi % 2
            c = pltpu.make_async_copy(
                wins[l].at[pl.ds(k * DK, DK), :], win_vm.at[slot],
                load_sems.at[0, slot])
            c.start()
            return c

        def issue_wout(oi):
            l, t = divmod(oi, NTO)
            slot = oi % 2
            c = pltpu.make_async_copy(
                wouts[l].at[pl.ds(t * TH, TH), :], wout_vm.at[slot],
                load_sems.at[1, slot])
            c.start()
            return c

        def ar_descs(l):
            return [
                pltpu.make_async_remote_copy(
                    src_ref=ysend,
                    dst_ref=ar_recv.at[l, d - 1],
                    send_sem=ar_send_sems.at[l, d - 1],
                    recv_sem=ar_recv_sems.at[l, d - 1],
                    device_id=((me + d) % N_DEV,),
                    device_id_type=pl.DeviceIdType.MESH,
                )
                for d in range(1, N_DEV)
            ]

        pend_w = {wi: issue_win(wi) for wi in range(2)}
        pend_o = {oi: issue_wout(oi) for oi in range(2)}
        ar = {}
        for l in range(NL):
            if l > 0:
                for r in ar[l - 1]:
                    r.wait_send()
                for r in ar[l - 1]:
                    r.wait_recv()
                xbuf[...] = (
                    ybuf[...]
                    + ar_recv[l - 1, 0].astype(jnp.float32)
                    + ar_recv[l - 1, 1].astype(jnp.float32)
                    + ar_recv[l - 1, 2].astype(jnp.float32))
            for k in range(NKW):
                wi = l * NKW + k
                c = pend_w.pop(wi)
                c.wait()
                xv = (x_ref if l == 0 else xbuf)[:, pl.ds(k * DK, DK)]
                hp = jnp.dot(xv, win_vm[wi % 2],
                             preferred_element_type=jnp.float32)
                if k == 0:
                    hbuf[...] = hp
                else:
                    hbuf[...] = hbuf[...] + hp
                if wi + 2 < NL * NKW:
                    pend_w[wi + 2] = issue_win(wi + 2)
            hbuf[...] = jnp.maximum(hbuf[...], 0.0)
            for t in range(NTO):
                oi = l * NTO + t
                c = pend_o.pop(oi)
                c.wait()
                hv = hbuf[:, pl.ds(t * TH, TH)]
                yp = jnp.dot(hv, wout_vm[oi % 2],
                             preferred_element_type=jnp.float32)
                if t == 0:
                    ybuf[...] = yp
                else:
                    ybuf[...] = ybuf[...] + yp
                if oi + 2 < NL * NTO:
                    pend_o[oi + 2] = issue_wout(oi + 2)
            ysend[...] = ybuf[...].astype(jnp.bfloat16)
            if l < NL - 1:
                descs = ar_descs(l)
                for r in descs:
                    r.start()
                ar[l] = descs
            else:
                rs = []
                for d in range(1, N_DEV):
                    j = (me + d) % N_DEV
                    r = pltpu.make_async_remote_copy(
                        src_ref=ysend.at[pl.ds(j * Bq, Bq), :],
                        dst_ref=rs_recv.at[d - 1],
                        send_sem=rs_send_sems.at[d - 1],
                        recv_sem=rs_recv_sems.at[d - 1],
                        device_id=(j,),
                        device_id_type=pl.DeviceIdType.MESH,
                    )
                    r.start()
                    rs.append(r)
                for r in rs:
                    r.wait_send()
                for r in rs:
                    r.wait_recv()
                out_ref[...] = (
                    ybuf[pl.ds(me * Bq, Bq), :]
                    + rs_recv[0].astype(jnp.float32)
                    + rs_recv[1].astype(jnp.float32)
                    + rs_recv[2].astype(jnp.float32))

    return pl.pallas_call(
        body,
        out_shape=jax.ShapeDtypeStruct((Bq, D), jnp.float32),
        in_specs=[
            pl.BlockSpec(memory_space=pltpu.MemorySpace.VMEM),
            pl.BlockSpec(memory_space=pl.ANY),
            pl.BlockSpec(memory_space=pl.ANY),
            pl.BlockSpec(memory_space=pl.ANY),
            pl.BlockSpec(memory_space=pl.ANY),
            pl.BlockSpec(memory_space=pl.ANY),
            pl.BlockSpec(memory_space=pl.ANY),
        ],
        out_specs=pl.BlockSpec(memory_space=pltpu.MemorySpace.VMEM),
        scratch_shapes=[
            pltpu.VMEM((B, D), jnp.float32),
            pltpu.VMEM((B, D), jnp.float32),
            pltpu.VMEM((B, H), jnp.float32),
            pltpu.VMEM((B, D), jnp.bfloat16),
            pltpu.VMEM((2, D // NKW, H), jnp.float32),
            pltpu.VMEM((2, H // NTO, D), jnp.float32),
            pltpu.VMEM((2, 3, B, D), jnp.bfloat16),
            pltpu.VMEM((3, Bq, D), jnp.bfloat16),
            pltpu.SemaphoreType.DMA((2, 2)),
            pltpu.SemaphoreType.DMA((2, 3)),
            pltpu.SemaphoreType.DMA((2, 3)),
            pltpu.SemaphoreType.DMA((3,)),
            pltpu.SemaphoreType.DMA((3,)),
        ],
        compiler_params=pltpu.CompilerParams(
            collective_id=0,
            vmem_limit_bytes=60 * 1024 * 1024,
        ),
    )(x, Win0, Wout0, Win1, Wout1, Win2, Wout2)
